# baseline (speedup 1.0000x reference)
import functools

import numpy as np
import jax
import jax.numpy as jnp
from jax import lax
from jax.experimental import pallas as pl
from jax.experimental.pallas import tpu as pltpu
from jax.experimental.pallas import tpu_sc as plsc

B = 4096
L = 200
D = 64
N_ROWS = B * L
NW = 32
PER_W = N_ROWS // NW
SUBG = 128
CHUNK = 512
NSUB = CHUNK // SUBG
N_CHUNKS = PER_W // CHUNK
PAIRS = N_CHUNKS // 2
BB = 64


def _positional_encoding():
    pos = np.arange(L, dtype=np.float32)[:, None]
    div = np.exp(np.arange(0, D, 2, dtype=np.float32) * (-np.log(10000.0) / D))
    pe = np.zeros((L, D), dtype=np.float32)
    pe[:, 0::2] = np.sin(pos * div)
    pe[:, 1::2] = np.cos(pos * div)
    return jnp.asarray(pe)


@functools.partial(
    pl.kernel,
    mesh=plsc.VectorSubcoreMesh(core_axis_name="c", subcore_axis_name="s"),
    compiler_params=pltpu.CompilerParams(use_tc_tiling_on_sc=False),
    out_type=jax.ShapeDtypeStruct((N_ROWS, D), jnp.float32),
    scratch_types=[
        pltpu.VMEM((PER_W,), jnp.int32),
        pltpu.VMEM((CHUNK, D), jnp.float32),
        pltpu.VMEM((CHUNK, D), jnp.float32),
        pltpu.SemaphoreType.DMA,
        pltpu.SemaphoreType.DMA,
    ],
)
def _sc_gather(seq_hbm, table_hbm, out_hbm, idx_v, rows0, rows1, sg0, sg1):
    nc = lax.axis_size("c")
    wid = lax.axis_index("s") * nc + lax.axis_index("c")
    row0 = wid * PER_W
    pltpu.sync_copy(seq_hbm.at[pl.ds(row0, PER_W)], idx_v)

    def issue_gather(c, rows_ref, sem):
        cc = lax.min(c, N_CHUNKS - 1)
        for j in range(NSUB):
            pltpu.async_copy(
                table_hbm.at[idx_v.at[pl.ds(cc * CHUNK + j * SUBG, SUBG)]],
                rows_ref.at[pl.ds(j * SUBG, SUBG)],
                sem,
            )

    def wait_gather(rows_ref, sem):
        pltpu.make_async_copy(
            out_hbm.at[pl.ds(0, CHUNK)], rows_ref, sem
        ).wait()

    def step(c, buf, nbuf, sem, nsem):
        issue_gather(c + 1, nbuf, nsem)
        wait_gather(buf, sem)
        pltpu.sync_copy(buf, out_hbm.at[pl.ds(row0 + c * CHUNK, CHUNK)])

    issue_gather(0, rows0, sg0)

    def pair_body(g, carry):
        step(2 * g, rows0, rows1, sg0, sg1)
        step(2 * g + 1, rows1, rows0, sg1, sg0)
        return carry

    lax.fori_loop(0, PAIRS, pair_body, 0)
    wait_gather(rows0, sg0)


def _tc_add_body(x_ref, pe_ref, o_ref):
    o_ref[...] = x_ref[...] + pe_ref[...][None, :, :]


_tc_add = pl.pallas_call(
    _tc_add_body,
    grid=(B // BB,),
    in_specs=[
        pl.BlockSpec((BB, L, D), lambda i: (i, 0, 0)),
        pl.BlockSpec((L, D), lambda i: (0, 0)),
    ],
    out_specs=pl.BlockSpec((BB, L, D), lambda i: (i, 0, 0)),
    out_shape=jax.ShapeDtypeStruct((B, L, D), jnp.float32),
)


def kernel(sequence, table):
    assert sequence.shape == (B, L), sequence.shape
    assert table.shape == (100001, D), table.shape
    seq1d = sequence.reshape(-1).astype(jnp.int32)
    gathered = _sc_gather(seq1d, table)
    pe = _positional_encoding()
    return _tc_add(gathered.reshape(B, L, D), pe)


# speedup vs baseline: 1.1402x; 1.1402x over previous
import functools

import numpy as np
import jax
import jax.numpy as jnp
from jax import lax
from jax.experimental import pallas as pl
from jax.experimental.pallas import tpu as pltpu
from jax.experimental.pallas import tpu_sc as plsc

B = 4096
L = 200
D = 64
N_ROWS = B * L
NW = 32
PER_W = N_ROWS // NW
SUBG = 128
CHUNK = 512
NSUB = CHUNK // SUBG
N_CHUNKS = PER_W // CHUNK
PAIRS = N_CHUNKS // 2
BB = 64


def _positional_encoding():
    pos = np.arange(L, dtype=np.float32)[:, None]
    div = np.exp(np.arange(0, D, 2, dtype=np.float32) * (-np.log(10000.0) / D))
    pe = np.zeros((L, D), dtype=np.float32)
    pe[:, 0::2] = np.sin(pos * div)
    pe[:, 1::2] = np.cos(pos * div)
    return jnp.asarray(pe)


@functools.partial(
    pl.kernel,
    mesh=plsc.VectorSubcoreMesh(core_axis_name="c", subcore_axis_name="s"),
    compiler_params=pltpu.CompilerParams(use_tc_tiling_on_sc=False),
    out_type=jax.ShapeDtypeStruct((N_ROWS, D), jnp.float32),
    scratch_types=[
        pltpu.VMEM((PER_W,), jnp.int32),
        pltpu.VMEM((CHUNK, D), jnp.float32),
        pltpu.VMEM((CHUNK, D), jnp.float32),
        pltpu.SemaphoreType.DMA,
        pltpu.SemaphoreType.DMA,
    ],
)
def _sc_gather(seq_hbm, table_hbm, out_hbm, idx_v, rows0, rows1, sg0, sg1):
    nc = lax.axis_size("c")
    wid = lax.axis_index("s") * nc + lax.axis_index("c")
    row0 = wid * PER_W
    pltpu.sync_copy(seq_hbm.at[pl.ds(row0, PER_W)], idx_v)

    def issue_gather(c, rows_ref, sem):
        cc = lax.min(c, N_CHUNKS - 1)
        for j in range(NSUB):
            pltpu.async_copy(
                table_hbm.at[idx_v.at[pl.ds(cc * CHUNK + j * SUBG, SUBG)]],
                rows_ref.at[pl.ds(j * SUBG, SUBG)],
                sem,
            )

    def wait_gather(rows_ref, sem):
        pltpu.make_async_copy(
            out_hbm.at[pl.ds(0, CHUNK)], rows_ref, sem
        ).wait()

    def step(c, buf, nbuf, sem, nsem):
        issue_gather(c + 1, nbuf, nsem)
        wait_gather(buf, sem)
        pltpu.sync_copy(buf, out_hbm.at[pl.ds(row0 + c * CHUNK, CHUNK)])

    issue_gather(0, rows0, sg0)

    def pair_body(g, carry):
        step(2 * g, rows0, rows1, sg0, sg1)
        step(2 * g + 1, rows1, rows0, sg1, sg0)
        return carry

    lax.fori_loop(0, PAIRS, pair_body, 0)
    wait_gather(rows0, sg0)


LP = L // 2


def _tc_add_body(x_ref, pe_ref, o_ref):
    xv = x_ref[...]
    pev = jnp.broadcast_to(pe_ref[...][None], (BB, LP, 2 * D))
    sv = xv + pev.reshape(BB * LP, 2 * D)
    halves = jnp.stack([sv[:, :D], sv[:, D:]], axis=1)
    o_ref[...] = halves.reshape(BB, L, D)


_tc_add = pl.pallas_call(
    _tc_add_body,
    grid=(B // BB,),
    in_specs=[
        pl.BlockSpec((BB * LP, 128), lambda i: (i, 0)),
        pl.BlockSpec((LP, 128), lambda i: (0, 0)),
    ],
    out_specs=pl.BlockSpec((BB, L, D), lambda i: (i, 0, 0)),
    out_shape=jax.ShapeDtypeStruct((B, L, D), jnp.float32),
)


def kernel(sequence, table):
    assert sequence.shape == (B, L), sequence.shape
    assert table.shape == (100001, D), table.shape
    seq1d = sequence.reshape(-1).astype(jnp.int32)
    gathered = _sc_gather(seq1d, table)
    x128 = gathered.reshape(N_ROWS // 2, 2 * D)
    pe128 = _positional_encoding().reshape(LP, 2 * D)
    return _tc_add(x128, pe128)
